# Initial kernel scaffold; baseline (speedup 1.0000x reference)
import jax
import jax.numpy as jnp
from jax.experimental import pallas as pl


def kernel(input, lit_vars, lit_pos, segment_ids):
    raise NotImplementedError("write your pallas kernel here")


# speedup vs baseline: 110.0891x; 110.0891x over previous
import functools

import jax
import jax.numpy as jnp
from jax import lax
from jax.experimental import pallas as pl
from jax.experimental.pallas import tpu as pltpu
from jax.experimental.pallas import tpu_sc as plsc

_NUM_VARS = 100000
_NUM_LITS = 3200000
_NUM_CLAUSES = 1000000

_NC = 2
_NS = 16
_NW = _NC * _NS
_L = 16

_C = _NUM_LITS // _NW
_B = 2000
_NBLK = _C // _B
_VPB = _B // _L

_F32_INF = float("inf")


def _vgather(x, idx):
    dnums = lax.GatherDimensionNumbers(
        offset_dims=(), collapsed_slice_dims=(0,), start_index_map=(0,))
    return lax.gather(x, idx[:, None], dnums, slice_sizes=(1,),
                      mode=lax.GatherScatterMode.PROMISE_IN_BOUNDS)


def _sc_body(inp_hbm, vars_hbm, pos_hbm, seg_hbm,
             rmin_out, hm_out, pm_out, pc_out,
             table_v, vars_v, pos_v, seg_v, prev_v, st_f, st_i):
    wid = lax.axis_index("s") * _NC + lax.axis_index("c")
    base = wid * _C

    pltpu.sync_copy(inp_hbm, table_v)

    poff = pl.multiple_of(jnp.maximum(base - _L, 0), 8)
    pltpu.sync_copy(seg_hbm.at[pl.ds(poff, _L)], prev_v)

    iota = lax.iota(jnp.int32, _L)
    zero_i = jnp.full((_L,), 0, jnp.int32)
    one_i = jnp.full((_L,), 1, jnp.int32)
    idx1 = jnp.maximum(iota - one_i, zero_i)
    idxp1 = jnp.minimum(iota + one_i, jnp.full((_L,), _L - 1, jnp.int32))
    c15 = jnp.full((_L,), _L - 1, jnp.int32)
    ind0 = iota == zero_i

    widv = jnp.broadcast_to(wid, (_L,)).astype(jnp.int32)
    pc0 = jnp.where(widv == zero_i, jnp.full((_L,), -1, jnp.int32),
                    _vgather(prev_v[...], c15))

    inf = jnp.full((_L,), _F32_INF)
    ninf = -inf
    half = jnp.full((_L,), 0.5, jnp.float32)
    one = jnp.full((_L,), 1.0, jnp.float32)
    two = jnp.full((_L,), 2.0, jnp.float32)
    zero = jnp.full((_L,), 0.0, jnp.float32)

    def vreg_step(i, carry):
        pc, pc1, pm, rminv, hmv = carry
        sl = pl.ds(i * _L, _L)
        v = vars_v[sl]
        p = pos_v[sl]
        s = seg_v[sl]
        tv = plsc.load_gather(table_v, [v])
        f = jnp.where(p == one_i, tv, one - tv)

        sprev = _vgather(s, idx1)
        b = s != sprev
        r = plsc.cumsum(jnp.where(b, one, zero))
        cm = plsc.cummax(r + half * f)
        scan = (cm - r) * two

        m = s == pc
        scan = jnp.where(m, jnp.maximum(scan, pm), scan)

        mh = s == pc0
        hmv = jnp.maximum(hmv, jnp.where(mh, f, ninf))

        snext = _vgather(s, idxp1)
        e = s != snext
        cand = jnp.where(e, scan, inf)
        nm0 = ind0 & jnp.logical_not(m)
        cand2 = jnp.where(nm0, pm, inf)
        gd = snext > s + one_i
        gp = ind0 & (s > pc1)
        cand = jnp.where(gd | gp, zero, cand)

        rminv = jnp.minimum(rminv, jnp.minimum(cand, cand2))

        pc_n = _vgather(s, c15)
        pm_n = _vgather(scan, c15)
        return pc_n, pc_n + one_i, pm_n, rminv, hmv

    def block_step(j, carry):
        start = pl.multiple_of(base + j * _B, 8)
        pltpu.sync_copy(vars_hbm.at[pl.ds(start, _B)], vars_v)
        pltpu.sync_copy(pos_hbm.at[pl.ds(start, _B)], pos_v)
        pltpu.sync_copy(seg_hbm.at[pl.ds(start, _B)], seg_v)
        return lax.fori_loop(0, _VPB, vreg_step, carry)

    init = (pc0, pc0 + one_i, inf, inf, ninf)
    pc, _, pm, rminv, hmv = lax.fori_loop(0, _NBLK, block_step, init)

    st_f[...] = rminv
    pltpu.sync_copy(st_f, rmin_out.at[wid])
    st_f[...] = hmv
    pltpu.sync_copy(st_f, hm_out.at[wid])
    st_f[...] = pm
    pltpu.sync_copy(st_f, pm_out.at[wid])
    st_i[...] = pc
    pltpu.sync_copy(st_i, pc_out.at[wid])


@jax.jit
def kernel(input, lit_vars, lit_pos, segment_ids):
    mesh = plsc.VectorSubcoreMesh(core_axis_name="c", subcore_axis_name="s",
                                  num_cores=_NC, num_subcores=_NS)
    f32 = jnp.float32
    run = pl.kernel(
        _sc_body,
        out_type=[
            jax.ShapeDtypeStruct((_NW, _L), f32),
            jax.ShapeDtypeStruct((_NW, _L), f32),
            jax.ShapeDtypeStruct((_NW, _L), f32),
            jax.ShapeDtypeStruct((_NW, _L), jnp.int32),
        ],
        mesh=mesh,
        compiler_params=pltpu.CompilerParams(needs_layout_passes=False),
        scratch_types=[
            pltpu.VMEM((_NUM_VARS,), f32),
            pltpu.VMEM((_B,), jnp.int32),
            pltpu.VMEM((_B,), jnp.int32),
            pltpu.VMEM((_B,), jnp.int32),
            pltpu.VMEM((_L,), jnp.int32),
            pltpu.VMEM((_L,), f32),
            pltpu.VMEM((_L,), jnp.int32),
        ],
    )
    rmin_a, hm_a, pm_a, pc_a = run(input, lit_vars, lit_pos, segment_ids)

    rmin_w = jnp.min(rmin_a, axis=1)
    hm_w = jnp.max(hm_a, axis=1)
    pm_w = pm_a[:, 0]
    pc_w = pc_a[:, 0]

    inf = _F32_INF
    gmin = inf
    cur_seg = jnp.int32(-1)
    cur_max = -inf
    for w in range(_NW):
        run_val = jnp.maximum(cur_max, hm_w[w])
        ended = pc_w[w] != cur_seg
        cand = jnp.where((cur_seg >= 0) & ended, run_val, inf)
        gmin = jnp.minimum(gmin, jnp.minimum(cand, rmin_w[w]))
        cur_max = jnp.where(ended, pm_w[w], run_val)
        cur_seg = pc_w[w]
    gmin = jnp.minimum(gmin, cur_max)
    gmin = jnp.where(pc_w[_NW - 1] < _NUM_CLAUSES - 1, jnp.float32(0.0), gmin)
    return gmin


# speedup vs baseline: 161.0912x; 1.4633x over previous
import functools

import jax
import jax.numpy as jnp
from jax import lax
from jax.experimental import pallas as pl
from jax.experimental.pallas import tpu as pltpu
from jax.experimental.pallas import tpu_sc as plsc

_NUM_VARS = 100000
_NUM_LITS = 3200000
_NUM_CLAUSES = 1000000

_NC = 2
_NS = 16
_NW = _NC * _NS
_L = 16

_C = _NUM_LITS // _NW
_B = 2000
_NBLK = _C // _B
_VPB = _B // _L

_F32_INF = float("inf")


def _vgather(x, idx):
    dnums = lax.GatherDimensionNumbers(
        offset_dims=(), collapsed_slice_dims=(0,), start_index_map=(0,))
    return lax.gather(x, idx[:, None], dnums, slice_sizes=(1,),
                      mode=lax.GatherScatterMode.PROMISE_IN_BOUNDS)


def _sc_body(inp_hbm, vars_hbm, pos_hbm, seg_hbm,
             rmin_out, hm_out, pm_out, pc_out,
             table_v, vars_v, pos_v, seg_v, vars_w, pos_w, seg_w,
             prev_v, st_f, st_i, sem0, sem1):
    wid = lax.axis_index("s") * _NC + lax.axis_index("c")
    base = wid * _C

    def issue(j, bufs, sem):
        start = pl.multiple_of(base + j * _B, 8)
        pltpu.make_async_copy(vars_hbm.at[pl.ds(start, _B)], bufs[0], sem).start()
        pltpu.make_async_copy(pos_hbm.at[pl.ds(start, _B)], bufs[1], sem).start()
        pltpu.make_async_copy(seg_hbm.at[pl.ds(start, _B)], bufs[2], sem).start()

    def drain(bufs, sem):
        pltpu.make_async_copy(vars_hbm.at[pl.ds(0, _B)], bufs[0], sem).wait()
        pltpu.make_async_copy(pos_hbm.at[pl.ds(0, _B)], bufs[1], sem).wait()
        pltpu.make_async_copy(seg_hbm.at[pl.ds(0, _B)], bufs[2], sem).wait()

    bufs0 = (vars_v, pos_v, seg_v)
    bufs1 = (vars_w, pos_w, seg_w)

    issue(0, bufs0, sem0)

    pltpu.sync_copy(inp_hbm, table_v)

    poff = pl.multiple_of(jnp.maximum(base - _L, 0), 8)
    pltpu.sync_copy(seg_hbm.at[pl.ds(poff, _L)], prev_v)

    iota = lax.iota(jnp.int32, _L)
    zero_i = jnp.full((_L,), 0, jnp.int32)
    one_i = jnp.full((_L,), 1, jnp.int32)
    idx1 = jnp.maximum(iota - one_i, zero_i)
    idxp1 = jnp.minimum(iota + one_i, jnp.full((_L,), _L - 1, jnp.int32))
    c15 = jnp.full((_L,), _L - 1, jnp.int32)
    ind0 = iota == zero_i

    widv = jnp.broadcast_to(wid, (_L,)).astype(jnp.int32)
    pc0 = jnp.where(widv == zero_i, jnp.full((_L,), -1, jnp.int32),
                    _vgather(prev_v[...], c15))

    inf = jnp.full((_L,), _F32_INF)
    ninf = -inf
    half = jnp.full((_L,), 0.5, jnp.float32)
    one = jnp.full((_L,), 1.0, jnp.float32)
    two = jnp.full((_L,), 2.0, jnp.float32)
    zero = jnp.full((_L,), 0.0, jnp.float32)

    def make_vreg_step(bufs):
      vb, pb, sb = bufs

      def vreg_step(i, carry):
        pc, pm, rminv, hmv = carry
        sl = pl.ds(i * _L, _L)
        v = vb[sl]
        p = pb[sl]
        s = sb[sl]
        tv = plsc.load_gather(table_v, [v])
        f = jnp.where(p == one_i, tv, one - tv)

        sprev = jnp.where(ind0, pc, _vgather(s, idx1))
        d = s - sprev
        b = d != zero_i

        r = plsc.cumsum(jnp.where(b, two, zero))
        cm = plsc.cummax(r + f)
        scan = cm - r

        m = s == pc
        scan = jnp.where(m, jnp.maximum(scan, pm), scan)

        mh = s == pc0
        hmv = jnp.maximum(hmv, jnp.where(mh, f, ninf))

        snext = _vgather(s, idxp1)
        e = s != snext
        cand = jnp.where(e, scan, inf)
        cand2 = jnp.where(ind0 & b, pm, inf)
        cand = jnp.where(d > one_i, zero, cand)

        rminv = jnp.minimum(rminv, jnp.minimum(cand, cand2))

        pc_n = _vgather(s, c15)
        pm_n = _vgather(scan, c15)
        return pc_n, pm_n, rminv, hmv

      return vreg_step

    step0 = make_vreg_step(bufs0)
    step1 = make_vreg_step(bufs1)

    def pair_step(t, carry):
        issue(2 * t + 1, bufs1, sem1)
        drain(bufs0, sem0)
        carry = lax.fori_loop(0, _VPB, step0, carry)

        @pl.when(t < _NBLK // 2 - 1)
        def _():
            issue(2 * t + 2, bufs0, sem0)

        drain(bufs1, sem1)
        return lax.fori_loop(0, _VPB, step1, carry)

    init = (pc0, inf, inf, ninf)
    pc, pm, rminv, hmv = lax.fori_loop(0, _NBLK // 2, pair_step, init)

    st_f[...] = rminv
    pltpu.sync_copy(st_f, rmin_out.at[wid])
    st_f[...] = hmv
    pltpu.sync_copy(st_f, hm_out.at[wid])
    st_f[...] = pm
    pltpu.sync_copy(st_f, pm_out.at[wid])
    st_i[...] = pc
    pltpu.sync_copy(st_i, pc_out.at[wid])


@jax.jit
def kernel(input, lit_vars, lit_pos, segment_ids):
    mesh = plsc.VectorSubcoreMesh(core_axis_name="c", subcore_axis_name="s",
                                  num_cores=_NC, num_subcores=_NS)
    f32 = jnp.float32
    run = pl.kernel(
        _sc_body,
        out_type=[
            jax.ShapeDtypeStruct((_NW, _L), f32),
            jax.ShapeDtypeStruct((_NW, _L), f32),
            jax.ShapeDtypeStruct((_NW, _L), f32),
            jax.ShapeDtypeStruct((_NW, _L), jnp.int32),
        ],
        mesh=mesh,
        compiler_params=pltpu.CompilerParams(needs_layout_passes=False),
        scratch_types=[
            pltpu.VMEM((_NUM_VARS,), f32),
            pltpu.VMEM((_B,), jnp.int32),
            pltpu.VMEM((_B,), jnp.int32),
            pltpu.VMEM((_B,), jnp.int32),
            pltpu.VMEM((_B,), jnp.int32),
            pltpu.VMEM((_B,), jnp.int32),
            pltpu.VMEM((_B,), jnp.int32),
            pltpu.VMEM((_L,), jnp.int32),
            pltpu.VMEM((_L,), f32),
            pltpu.VMEM((_L,), jnp.int32),
            pltpu.SemaphoreType.DMA,
            pltpu.SemaphoreType.DMA,
        ],
    )
    rmin_a, hm_a, pm_a, pc_a = run(input, lit_vars, lit_pos, segment_ids)

    rmin_w = jnp.min(rmin_a, axis=1)
    hm_w = jnp.max(hm_a, axis=1)
    pm_w = pm_a[:, 0]
    pc_w = pc_a[:, 0]

    inf = _F32_INF
    gmin = inf
    cur_seg = jnp.int32(-1)
    cur_max = -inf
    for w in range(_NW):
        run_val = jnp.maximum(cur_max, hm_w[w])
        ended = pc_w[w] != cur_seg
        cand = jnp.where((cur_seg >= 0) & ended, run_val, inf)
        gmin = jnp.minimum(gmin, jnp.minimum(cand, rmin_w[w]))
        cur_max = jnp.where(ended, pm_w[w], run_val)
        cur_seg = pc_w[w]
    gmin = jnp.minimum(gmin, cur_max)
    gmin = jnp.where(pc_w[_NW - 1] < _NUM_CLAUSES - 1, jnp.float32(0.0), gmin)
    return gmin
